# baseline (speedup 1.0000x reference)
import jax
import jax.numpy as jnp
from jax import lax
from jax.experimental import pallas as pl
from jax.experimental.pallas import tpu as pltpu
from jax.experimental.pallas import tpu_sc as plsc

N_NODES = 10000
N_EDGES = 320000
D = 128

NC = 2
NS = 16
NW = NC * NS
EDGES_PER_WORKER = N_EDGES // NW
CHUNK = 80
NCHUNKS = EDGES_PER_WORKER // CHUNK

N_PAD = 10240
ROWS_PER_TILE = N_PAD // NS
STAGE_CHUNKS = ROWS_PER_TILE // CHUNK


def _sc_body(feat_hbm, src_hbm, dst3_hbm, iota3_hbm, zrows_hbm, ones_hbm,
             aggp_hbm, degp_hbm,
             acc_sh, iidx_all, sidx0_v, sidx1_v, didx_all,
             rows0_v, rows1_v, sem0, sem1, sem2, sem3):
    c = lax.axis_index("c")
    s = lax.axis_index("s")
    wid = s * NC + c
    row0 = s * ROWS_PER_TILE
    base = wid * EDGES_PER_WORKER

    pltpu.sync_copy(dst3_hbm.at[wid], didx_all)
    pltpu.sync_copy(iota3_hbm.at[s], iidx_all)

    def zero_own_rows():
        pltpu.sync_copy(zrows_hbm, rows0_v)
        for j in range(STAGE_CHUNKS):
            pltpu.sync_copy(rows0_v, acc_sh.at[iidx_all.at[j]])

    def copy_out(dst_hbm_ref):
        for j in range(STAGE_CHUNKS):
            off = row0 + j * CHUNK
            pltpu.async_copy(acc_sh.at[iidx_all.at[j]], rows0_v, sem0).wait()
            pltpu.sync_copy(rows0_v, dst_hbm_ref.at[c, pl.ds(off, CHUNK)])

    zero_own_rows()
    plsc.subcore_barrier()

    pltpu.sync_copy(src_hbm.at[pl.ds(base, CHUNK)], sidx0_v)
    g0 = pltpu.async_copy(feat_hbm.at[sidx0_v], rows0_v, sem0)

    def p1_body(i, carry):
        g = 2 * i
        off = base + g * CHUNK
        pltpu.sync_copy(src_hbm.at[pl.ds(off + CHUNK, CHUNK)], sidx1_v)
        pltpu.async_copy(feat_hbm.at[sidx1_v], rows1_v, sem1)
        pltpu.make_async_copy(feat_hbm.at[sidx0_v], rows0_v, sem0).wait()
        pltpu.sync_copy(rows0_v, acc_sh.at[didx_all.at[g]], add=True)
        pltpu.sync_copy(src_hbm.at[pl.ds(off + 2 * CHUNK, CHUNK)], sidx0_v)
        pltpu.async_copy(feat_hbm.at[sidx0_v], rows0_v, sem0)
        pltpu.make_async_copy(feat_hbm.at[sidx1_v], rows1_v, sem1).wait()
        pltpu.sync_copy(rows1_v, acc_sh.at[didx_all.at[g + 1]], add=True)
        return carry

    lax.fori_loop(0, (NCHUNKS - 1) // 2, p1_body, 0)
    g0.wait()
    pltpu.sync_copy(rows0_v, acc_sh.at[didx_all.at[NCHUNKS - 1]], add=True)
    plsc.subcore_barrier()
    copy_out(aggp_hbm)

    pltpu.sync_copy(ones_hbm, rows1_v)
    plsc.subcore_barrier()

    def fire(k, sem):
        return pltpu.async_copy(rows1_v, acc_sh.at[didx_all.at[k]], sem,
                                add=True)

    def drain(sem):
        pltpu.make_async_copy(rows1_v, acc_sh.at[didx_all.at[0]], sem).wait()

    fire(0, sem0)
    fire(1, sem1)
    fire(2, sem2)

    def p2_body(i, carry):
        g = 3 + 4 * i
        fire(g, sem3)
        drain(sem0)
        fire(g + 1, sem0)
        drain(sem1)
        fire(g + 2, sem1)
        drain(sem2)
        fire(g + 3, sem2)
        drain(sem3)
        return carry

    lax.fori_loop(0, (NCHUNKS - 5) // 4, p2_body, 0)
    fire(NCHUNKS - 2, sem3)
    drain(sem0)
    fire(NCHUNKS - 1, sem0)
    drain(sem1)
    drain(sem2)
    drain(sem3)
    drain(sem0)
    plsc.subcore_barrier()
    copy_out(degp_hbm)


_sc_segment_sum = pl.kernel(
    _sc_body,
    out_type=(
        jax.ShapeDtypeStruct((NC, N_PAD, D), jnp.float32),
        jax.ShapeDtypeStruct((NC, N_PAD, D), jnp.float32),
    ),
    mesh=plsc.VectorSubcoreMesh(core_axis_name="c", subcore_axis_name="s"),
    scratch_types=[
        pltpu.VMEM_SHARED((N_PAD, D), jnp.float32),
        pltpu.VMEM((STAGE_CHUNKS, CHUNK), jnp.int32),
        pltpu.VMEM((CHUNK,), jnp.int32),
        pltpu.VMEM((CHUNK,), jnp.int32),
        pltpu.VMEM((NCHUNKS, CHUNK), jnp.int32),
        pltpu.VMEM((CHUNK, D), jnp.float32),
        pltpu.VMEM((CHUNK, D), jnp.float32),
        pltpu.SemaphoreType.DMA,
        pltpu.SemaphoreType.DMA,
        pltpu.SemaphoreType.DMA,
        pltpu.SemaphoreType.DMA,
    ],
)


BLK = 1000


def _tc_dense_body(feat_ref, w2_ref, b_ref, f_ref):
    f_ref[...] = (
        jnp.dot(feat_ref[...], w2_ref[...], preferred_element_type=jnp.float32)
        + b_ref[...]
    )


_tc_dense = pl.pallas_call(
    _tc_dense_body,
    grid=(N_NODES // BLK,),
    in_specs=[
        pl.BlockSpec((BLK, D), lambda i: (i, 0)),
        pl.BlockSpec((D, D), lambda i: (0, 0)),
        pl.BlockSpec((1, D), lambda i: (0, 0)),
    ],
    out_specs=pl.BlockSpec((BLK, D), lambda i: (i, 0)),
    out_shape=jax.ShapeDtypeStruct((N_NODES, D), jnp.float32),
)


def _tc_body(aggp_ref, degp_ref, f_ref, w1_ref, out_ref):
    agg = aggp_ref[0] + aggp_ref[1]
    deg = (degp_ref[0] + degp_ref[1]) - agg
    inv = 1.0 / jnp.maximum(deg[:, 0:1], 1.0)
    h = agg * inv
    out_ref[...] = (
        jnp.dot(h, w1_ref[...], preferred_element_type=jnp.float32)
        + f_ref[...]
    )


_tc_combine = pl.pallas_call(
    _tc_body,
    grid=(N_NODES // BLK,),
    in_specs=[
        pl.BlockSpec((NC, BLK, D), lambda i: (0, i, 0)),
        pl.BlockSpec((NC, BLK, D), lambda i: (0, i, 0)),
        pl.BlockSpec((BLK, D), lambda i: (i, 0)),
        pl.BlockSpec((D, D), lambda i: (0, 0)),
    ],
    out_specs=pl.BlockSpec((BLK, D), lambda i: (i, 0)),
    out_shape=jax.ShapeDtypeStruct((N_NODES, D), jnp.float32),
)


@jax.jit
def kernel(feat, edge_index, W, b):
    src = edge_index[0]
    dst3 = edge_index[1].reshape(NW, NCHUNKS, CHUNK)
    iota3 = jnp.arange(N_PAD, dtype=jnp.int32).reshape(NS, STAGE_CHUNKS, CHUNK)
    zrows = jnp.zeros((CHUNK, D), jnp.float32)
    ones = jnp.ones((CHUNK, D), jnp.float32)
    wt = W.T
    b2 = b.reshape(1, D)
    f = _tc_dense(feat, wt[D:2 * D], b2)
    aggp, degp = _sc_segment_sum(feat, src, dst3, iota3, zrows, ones)
    return _tc_combine(aggp, degp, f, wt[0:D])


# speedup vs baseline: 1.0060x; 1.0060x over previous
import jax
import jax.numpy as jnp
from jax import lax
from jax.experimental import pallas as pl
from jax.experimental.pallas import tpu as pltpu
from jax.experimental.pallas import tpu_sc as plsc

N_NODES = 10000
N_EDGES = 320000
D = 128

NC = 2
NS = 16
NW = NC * NS
EDGES_PER_WORKER = N_EDGES // NW
CHUNK = 80
NCHUNKS = EDGES_PER_WORKER // CHUNK

N_PAD = 10240
ROWS_PER_TILE = N_PAD // NS
STAGE_CHUNKS = ROWS_PER_TILE // CHUNK


def _sc_body(feat_hbm, src_hbm, dst3_hbm, iota3_hbm, zrows_hbm, ones_hbm,
             aggp_hbm, degp_hbm,
             acc_sh, iidx_all, sidx0_v, sidx1_v, didx_all,
             rows0_v, rows1_v, sem0, sem1, sem2, sem3):
    c = lax.axis_index("c")
    s = lax.axis_index("s")
    wid = s * NC + c
    row0 = s * ROWS_PER_TILE
    base = wid * EDGES_PER_WORKER

    pltpu.sync_copy(dst3_hbm.at[wid], didx_all)
    pltpu.sync_copy(iota3_hbm.at[s], iidx_all)

    def zero_own_rows():
        pltpu.sync_copy(zrows_hbm, rows0_v)
        for j in range(STAGE_CHUNKS):
            pltpu.sync_copy(rows0_v, acc_sh.at[iidx_all.at[j]])

    def copy_out(dst_hbm_ref):
        for j in range(STAGE_CHUNKS):
            off = row0 + j * CHUNK
            pltpu.async_copy(acc_sh.at[iidx_all.at[j]], rows0_v, sem0).wait()
            pltpu.sync_copy(rows0_v, dst_hbm_ref.at[c, pl.ds(off, CHUNK)])

    zero_own_rows()
    plsc.subcore_barrier()

    pltpu.sync_copy(src_hbm.at[pl.ds(base, CHUNK)], sidx0_v)
    g0 = pltpu.async_copy(feat_hbm.at[sidx0_v], rows0_v, sem0)

    def p1_body(i, carry):
        g = 2 * i
        off = base + g * CHUNK
        pltpu.sync_copy(src_hbm.at[pl.ds(off + CHUNK, CHUNK)], sidx1_v)
        pltpu.async_copy(feat_hbm.at[sidx1_v], rows1_v, sem1)
        pltpu.make_async_copy(feat_hbm.at[sidx0_v], rows0_v, sem0).wait()
        pltpu.sync_copy(rows0_v, acc_sh.at[didx_all.at[g]], add=True)
        pltpu.sync_copy(src_hbm.at[pl.ds(off + 2 * CHUNK, CHUNK)], sidx0_v)
        pltpu.async_copy(feat_hbm.at[sidx0_v], rows0_v, sem0)
        pltpu.make_async_copy(feat_hbm.at[sidx1_v], rows1_v, sem1).wait()
        pltpu.sync_copy(rows1_v, acc_sh.at[didx_all.at[g + 1]], add=True)
        return carry

    lax.fori_loop(0, (NCHUNKS - 1) // 2, p1_body, 0)
    g0.wait()
    pltpu.sync_copy(rows0_v, acc_sh.at[didx_all.at[NCHUNKS - 1]], add=True)
    plsc.subcore_barrier()
    copy_out(aggp_hbm)

    pltpu.sync_copy(ones_hbm, rows1_v)
    plsc.subcore_barrier()

    def fire(k, sem):
        return pltpu.async_copy(rows1_v, acc_sh.at[didx_all.at[k]], sem,
                                add=True)

    def drain(sem):
        pltpu.make_async_copy(rows1_v, acc_sh.at[didx_all.at[0]], sem).wait()

    fire(0, sem0)
    fire(1, sem1)
    fire(2, sem2)

    def p2_body(i, carry):
        g = 3 + 4 * i
        fire(g, sem3)
        drain(sem0)
        fire(g + 1, sem0)
        drain(sem1)
        fire(g + 2, sem1)
        drain(sem2)
        fire(g + 3, sem2)
        drain(sem3)
        return carry

    lax.fori_loop(0, (NCHUNKS - 5) // 4, p2_body, 0)
    fire(NCHUNKS - 2, sem3)
    drain(sem0)
    fire(NCHUNKS - 1, sem0)
    drain(sem1)
    drain(sem2)
    drain(sem3)
    drain(sem0)
    plsc.subcore_barrier()
    copy_out(degp_hbm)


_sc_segment_sum = pl.kernel(
    _sc_body,
    out_type=(
        jax.ShapeDtypeStruct((NC, N_PAD, D), jnp.float32),
        jax.ShapeDtypeStruct((NC, N_PAD, D), jnp.float32),
    ),
    mesh=plsc.VectorSubcoreMesh(core_axis_name="c", subcore_axis_name="s"),
    scratch_types=[
        pltpu.VMEM_SHARED((N_PAD, D), jnp.float32),
        pltpu.VMEM((STAGE_CHUNKS, CHUNK), jnp.int32),
        pltpu.VMEM((CHUNK,), jnp.int32),
        pltpu.VMEM((CHUNK,), jnp.int32),
        pltpu.VMEM((NCHUNKS, CHUNK), jnp.int32),
        pltpu.VMEM((CHUNK, D), jnp.float32),
        pltpu.VMEM((CHUNK, D), jnp.float32),
        pltpu.SemaphoreType.DMA,
        pltpu.SemaphoreType.DMA,
        pltpu.SemaphoreType.DMA,
        pltpu.SemaphoreType.DMA,
    ],
)


BLK = 1000


def _tc_body(aggp_ref, degp_ref, feat_ref, wt_ref, b_ref, out_ref):
    agg = aggp_ref[0] + aggp_ref[1]
    deg = (degp_ref[0] + degp_ref[1]) - agg
    inv = 1.0 / jnp.maximum(deg[:, 0:1], 1.0)
    h = agg * inv
    w1 = wt_ref[0:D]
    w2 = wt_ref[D:2 * D]
    out_ref[...] = (
        jnp.dot(h, w1, preferred_element_type=jnp.float32)
        + jnp.dot(feat_ref[...], w2, preferred_element_type=jnp.float32)
        + b_ref[...]
    )


_tc_linear = pl.pallas_call(
    _tc_body,
    grid=(N_NODES // BLK,),
    in_specs=[
        pl.BlockSpec((NC, BLK, D), lambda i: (0, i, 0)),
        pl.BlockSpec((NC, BLK, D), lambda i: (0, i, 0)),
        pl.BlockSpec((BLK, D), lambda i: (i, 0)),
        pl.BlockSpec((2 * D, D), lambda i: (0, 0)),
        pl.BlockSpec((1, D), lambda i: (0, 0)),
    ],
    out_specs=pl.BlockSpec((BLK, D), lambda i: (i, 0)),
    out_shape=jax.ShapeDtypeStruct((N_NODES, D), jnp.float32),
)


@jax.jit
def kernel(feat, edge_index, W, b):
    src = edge_index[0]
    dst3 = edge_index[1].reshape(NW, NCHUNKS, CHUNK)
    iota3 = jnp.arange(N_PAD, dtype=jnp.int32).reshape(NS, STAGE_CHUNKS, CHUNK)
    zrows = jnp.zeros((CHUNK, D), jnp.float32)
    ones = jnp.ones((CHUNK, D), jnp.float32)
    aggp, degp = _sc_segment_sum(feat, src, dst3, iota3, zrows, ones)
    wt = W.T
    b2 = b.reshape(1, D)
    return _tc_linear(aggp, degp, feat, wt, b2)
